# initial kernel scaffold (unmeasured)
import jax
import jax.numpy as jnp
from jax import lax
from jax.experimental import pallas as pl
from jax.experimental.pallas import tpu as pltpu


def kernel(
    x,
):
    def body(*refs):
        pass

    out_shape = jax.ShapeDtypeStruct(..., jnp.float32)
    return pl.pallas_call(body, out_shape=out_shape)(...)


# baseline (device time: 409085 ns/iter reference)
import jax
import jax.numpy as jnp
from jax import lax
from jax.experimental import pallas as pl
from jax.experimental.pallas import tpu as pltpu

N_DEV = 16


def kernel(x):
    m_per, n = x.shape

    def body(x_ref, out_ref, send_sems, recv_sems):
        my_pos = lax.axis_index("i")
        right = lax.rem(my_pos + 1, N_DEV)

        out_ref[pl.ds(my_pos * m_per, m_per), :] = x_ref[:, :]

        for h in range(N_DEV - 1):
            src_origin = lax.rem(my_pos - h + N_DEV, N_DEV)
            rdma = pltpu.make_async_remote_copy(
                src_ref=out_ref.at[pl.ds(src_origin * m_per, m_per), :],
                dst_ref=out_ref.at[pl.ds(src_origin * m_per, m_per), :],
                send_sem=send_sems.at[h],
                recv_sem=recv_sems.at[h],
                device_id=(right,),
                device_id_type=pl.DeviceIdType.MESH,
            )
            rdma.start()
            rdma.wait()

    return pl.pallas_call(
        body,
        out_shape=jax.ShapeDtypeStruct((N_DEV * m_per, n), x.dtype),
        in_specs=[pl.BlockSpec(memory_space=pltpu.VMEM)],
        out_specs=pl.BlockSpec(memory_space=pltpu.VMEM),
        scratch_shapes=[
            pltpu.SemaphoreType.DMA((N_DEV - 1,)),
            pltpu.SemaphoreType.DMA((N_DEV - 1,)),
        ],
    )(x)


# device time: 225147 ns/iter; 1.8170x vs baseline; 1.8170x over previous
import jax
import jax.numpy as jnp
from jax import lax
from jax.experimental import pallas as pl
from jax.experimental.pallas import tpu as pltpu

N_DEV = 16
N_HOP = 8


def kernel(x):
    m_per, n = x.shape
    half = m_per // 2

    def body(x_ref, out_ref, cw_send, cw_recv, ccw_send, ccw_recv):
        my_pos = lax.axis_index("i")
        right = lax.rem(my_pos + 1, N_DEV)
        left = lax.rem(my_pos - 1 + N_DEV, N_DEV)

        barrier_sem = pltpu.get_barrier_semaphore()
        for nbr in (left, right):
            pl.semaphore_signal(
                barrier_sem, inc=1,
                device_id=(nbr,), device_id_type=pl.DeviceIdType.MESH,
            )
        pl.semaphore_wait(barrier_sem, 2)

        out_ref[pl.ds(my_pos * m_per, m_per), :] = x_ref[:, :]

        def cw_desc(h):
            origin = lax.rem(my_pos - h + N_DEV, N_DEV)
            if h < N_HOP - 1:
                rows = pl.ds(origin * m_per, m_per)
            else:
                rows = pl.ds(origin * m_per, half)
            return pltpu.make_async_remote_copy(
                src_ref=out_ref.at[rows, :],
                dst_ref=out_ref.at[rows, :],
                send_sem=cw_send.at[h],
                recv_sem=cw_recv.at[h],
                device_id=(right,),
                device_id_type=pl.DeviceIdType.MESH,
            )

        def ccw_desc(h):
            origin = lax.rem(my_pos + h, N_DEV)
            if h < N_HOP - 1:
                rows = pl.ds(origin * m_per, m_per)
            else:
                rows = pl.ds(origin * m_per + half, half)
            return pltpu.make_async_remote_copy(
                src_ref=out_ref.at[rows, :],
                dst_ref=out_ref.at[rows, :],
                send_sem=ccw_send.at[h],
                recv_sem=ccw_recv.at[h],
                device_id=(left,),
                device_id_type=pl.DeviceIdType.MESH,
            )

        cw = [cw_desc(h) for h in range(N_HOP)]
        ccw = [ccw_desc(h) for h in range(N_HOP)]

        cw[0].start()
        ccw[0].start()
        for h in range(1, N_HOP):
            cw[h - 1].wait_recv()
            cw[h].start()
            ccw[h - 1].wait_recv()
            ccw[h].start()
        cw[N_HOP - 1].wait_recv()
        ccw[N_HOP - 1].wait_recv()
        for h in range(N_HOP):
            cw[h].wait_send()
            ccw[h].wait_send()

    return pl.pallas_call(
        body,
        out_shape=jax.ShapeDtypeStruct((N_DEV * m_per, n), x.dtype),
        in_specs=[pl.BlockSpec(memory_space=pltpu.VMEM)],
        out_specs=pl.BlockSpec(memory_space=pltpu.VMEM),
        scratch_shapes=[
            pltpu.SemaphoreType.DMA((N_HOP,)),
            pltpu.SemaphoreType.DMA((N_HOP,)),
            pltpu.SemaphoreType.DMA((N_HOP,)),
            pltpu.SemaphoreType.DMA((N_HOP,)),
        ],
        compiler_params=pltpu.CompilerParams(collective_id=0),
    )(x)


# device time: 212580 ns/iter; 1.9244x vs baseline; 1.0591x over previous
import jax
import jax.numpy as jnp
from jax import lax
from jax.experimental import pallas as pl
from jax.experimental.pallas import tpu as pltpu

N_DEV = 16
N_HOP = 8


def kernel(x):
    m_per, n = x.shape
    half = m_per // 2

    def body(x_ref, out_ref, cw_send, cw_recv, ccw_send, ccw_recv):
        my_pos = lax.axis_index("i")
        right = lax.rem(my_pos + 1, N_DEV)
        left = lax.rem(my_pos - 1 + N_DEV, N_DEV)

        barrier_sem = pltpu.get_barrier_semaphore()
        for nbr in (left, right):
            pl.semaphore_signal(
                barrier_sem, inc=1,
                device_id=(nbr,), device_id_type=pl.DeviceIdType.MESH,
            )
        pl.semaphore_wait(barrier_sem, 2)

        out_ref[pl.ds(my_pos * m_per, m_per), :] = x_ref[:, :]

        def desc(h, p, sgn, send_sems, recv_sems, nbr):
            origin = lax.rem(my_pos + sgn * h + N_DEV, N_DEV)
            rows = pl.ds(origin * m_per + p * half, half)
            return pltpu.make_async_remote_copy(
                src_ref=out_ref.at[rows, :],
                dst_ref=out_ref.at[rows, :],
                send_sem=send_sems.at[h, p],
                recv_sem=recv_sems.at[h, p],
                device_id=(nbr,),
                device_id_type=pl.DeviceIdType.MESH,
            )

        cw = {(h, p): desc(h, p, -1, cw_send, cw_recv, right)
              for h in range(N_HOP) for p in (0, 1)
              if not (h == N_HOP - 1 and p == 1)}
        ccw = {(h, p): desc(h, p, +1, ccw_send, ccw_recv, left)
               for h in range(N_HOP) for p in (0, 1)
               if not (h == N_HOP - 1 and p == 0)}

        for p in (0, 1):
            cw[0, p].start()
            ccw[0, p].start()
        for h in range(1, N_HOP - 1):
            for p in (0, 1):
                cw[h - 1, p].wait_recv()
                cw[h, p].start()
                ccw[h - 1, p].wait_recv()
                ccw[h, p].start()
        cw[N_HOP - 2, 0].wait_recv()
        cw[N_HOP - 1, 0].start()
        ccw[N_HOP - 2, 1].wait_recv()
        ccw[N_HOP - 1, 1].start()
        cw[N_HOP - 2, 1].wait_recv()
        ccw[N_HOP - 2, 0].wait_recv()
        cw[N_HOP - 1, 0].wait_recv()
        ccw[N_HOP - 1, 1].wait_recv()
        for k in cw:
            cw[k].wait_send()
        for k in ccw:
            ccw[k].wait_send()

    return pl.pallas_call(
        body,
        out_shape=jax.ShapeDtypeStruct((N_DEV * m_per, n), x.dtype),
        in_specs=[pl.BlockSpec(memory_space=pltpu.VMEM)],
        out_specs=pl.BlockSpec(memory_space=pltpu.VMEM),
        scratch_shapes=[
            pltpu.SemaphoreType.DMA((N_HOP, 2)),
            pltpu.SemaphoreType.DMA((N_HOP, 2)),
            pltpu.SemaphoreType.DMA((N_HOP, 2)),
            pltpu.SemaphoreType.DMA((N_HOP, 2)),
        ],
        compiler_params=pltpu.CompilerParams(collective_id=0),
    )(x)


# device time: 180168 ns/iter; 2.2706x vs baseline; 1.1799x over previous
import jax
import jax.numpy as jnp
from jax import lax
from jax.experimental import pallas as pl
from jax.experimental.pallas import tpu as pltpu

N_DEV = 16
P = 4
Z = 4


def kernel(x):
    m_per, n = x.shape
    half = m_per // 2

    def body(x_ref, out_ref,
             zcw_s, zcw_r, zccw_s, zccw_r,
             pcwf_s, pcwf_r, pcwh_s, pcwh_r,
             pccwf_s, pccwf_r, pccwh_s, pccwh_r):
        my_pos = lax.axis_index("i")
        q = lax.rem(my_pos, P)
        zb = my_pos - q
        right = zb + lax.rem(q + 1, P)
        left = zb + lax.rem(q + 3, P)
        zup = lax.rem(my_pos + P, N_DEV)
        zdn = lax.rem(my_pos + (Z - 1) * P, N_DEV)

        def pos(dz, qq):
            return lax.rem(zb + P * dz, N_DEV) + qq

        def full_rows(p_):
            return pl.ds(p_ * m_per, m_per)

        def top_rows(p_):
            return pl.ds(p_ * m_per, half)

        def bot_rows(p_):
            return pl.ds(p_ * m_per + half, half)

        def desc(rows, ssem, rsem, target):
            return pltpu.make_async_remote_copy(
                src_ref=out_ref.at[rows, :],
                dst_ref=out_ref.at[rows, :],
                send_sem=ssem,
                recv_sem=rsem,
                device_id=(target,),
                device_id_type=pl.DeviceIdType.MESH,
            )

        barrier_sem = pltpu.get_barrier_semaphore()
        for nbr in (left, right, zup, zdn):
            pl.semaphore_signal(
                barrier_sem, inc=1,
                device_id=(nbr,), device_id_type=pl.DeviceIdType.MESH,
            )
        pl.semaphore_wait(barrier_sem, 4)

        out_ref[full_rows(my_pos), :] = x_ref[:, :]

        zcw0 = desc(full_rows(my_pos), zcw_s.at[0], zcw_r.at[0], zup)
        zcw1 = desc(top_rows(pos(Z - 1, q)), zcw_s.at[1], zcw_r.at[1], zup)
        zccw0 = desc(full_rows(my_pos), zccw_s.at[0], zccw_r.at[0], zdn)
        zccw1 = desc(bot_rows(pos(1, q)), zccw_s.at[1], zccw_r.at[1], zdn)

        ql = lax.rem(q + 3, P)
        qr = lax.rem(q + 1, P)
        pcw_f = [desc(full_rows(pos(dz, q)), pcwf_s.at[dz], pcwf_r.at[dz],
                      right) for dz in range(Z)]
        pcw_h = [desc(top_rows(pos(dz, ql)), pcwh_s.at[dz], pcwh_r.at[dz],
                      right) for dz in range(Z)]
        pccw_f = [desc(full_rows(pos(dz, q)), pccwf_s.at[dz], pccwf_r.at[dz],
                       left) for dz in range(Z)]
        pccw_h = [desc(bot_rows(pos(dz, qr)), pccwh_s.at[dz], pccwh_r.at[dz],
                       left) for dz in range(Z)]

        zcw0.start()
        zccw0.start()
        pcw_f[0].start()
        pccw_f[0].start()
        zcw0.wait_recv()
        zcw1.start()
        pcw_f[Z - 1].start()
        pccw_f[Z - 1].start()
        zccw0.wait_recv()
        zccw1.start()
        pcw_f[1].start()
        pccw_f[1].start()
        pcw_f[0].wait_recv()
        pcw_h[0].start()
        pccw_f[0].wait_recv()
        pccw_h[0].start()
        zcw1.wait_recv()
        zccw1.wait_recv()
        pcw_f[2].start()
        pccw_f[2].start()
        for dz in (Z - 1, 1, 2):
            pcw_f[dz].wait_recv()
            pcw_h[dz].start()
            pccw_f[dz].wait_recv()
            pccw_h[dz].start()
        for dz in range(Z):
            pcw_h[dz].wait_recv()
            pccw_h[dz].wait_recv()
        for d in (zcw0, zcw1, zccw0, zccw1, *pcw_f, *pcw_h,
                  *pccw_f, *pccw_h):
            d.wait_send()

    dma2 = pltpu.SemaphoreType.DMA((2,))
    dma4 = pltpu.SemaphoreType.DMA((Z,))
    return pl.pallas_call(
        body,
        out_shape=jax.ShapeDtypeStruct((N_DEV * m_per, n), x.dtype),
        in_specs=[pl.BlockSpec(memory_space=pltpu.VMEM)],
        out_specs=pl.BlockSpec(memory_space=pltpu.VMEM),
        scratch_shapes=[
            dma2, dma2, dma2, dma2,
            dma4, dma4, dma4, dma4,
            dma4, dma4, dma4, dma4,
        ],
        compiler_params=pltpu.CompilerParams(collective_id=0),
    )(x)
